# baseline (speedup 1.0000x reference)
import functools

import jax
import jax.numpy as jnp
from jax import lax
from jax.experimental import pallas as pl
from jax.experimental.pallas import tpu as pltpu
from jax.experimental.pallas import tpu_sc as plsc

_NC = 2
_NS = 16
_NW = _NC * _NS


@functools.lru_cache(maxsize=None)
def _gather_call(B, E, CH, BB, HW):
    per_w = B // _NW
    n_chunks = per_w // CH
    assert n_chunks >= 4 and n_chunks % 2 == 0
    mesh = plsc.VectorSubcoreMesh(core_axis_name="c", subcore_axis_name="s")

    Q = 128 // E
    YB = BB // Q

    def _out_slice(out_hbm, off):
        h = off // HW
        bo = off % HW
        blk = bo // BB
        t = bo % BB
        q = t // YB
        y0 = t % YB
        u0 = h * (HW // Q) + blk * YB + y0
        return out_hbm.at[pl.ds(u0, CH), pl.ds(q * E, E)]

    @functools.partial(
        pl.kernel,
        mesh=mesh,
        out_type=jax.ShapeDtypeStruct((B // Q, 128), jnp.float32),
        scratch_types=[
            pltpu.VMEM((CH,), jnp.int32),
            pltpu.VMEM((CH,), jnp.int32),
            pltpu.VMEM((CH, E), jnp.float32),
            pltpu.VMEM((CH, E), jnp.float32),
            pltpu.SemaphoreType.DMA,
            pltpu.SemaphoreType.DMA,
            pltpu.SemaphoreType.DMA,
            pltpu.SemaphoreType.DMA,
        ],
        compiler_params=pltpu.CompilerParams(use_tc_tiling_on_sc=False),
    )
    def k(idx_hbm, table_hbm, out_hbm, iv0, iv1, r0, r1, g0, g1, o0, o1):
        wid = lax.axis_index("s") * _NC + lax.axis_index("c")
        base = wid * per_w
        iv = (iv0, iv1)
        rows = (r0, r1)
        g = (g0, g1)
        o = (o0, o1)

        for b in range(2):
            pltpu.sync_copy(idx_hbm.at[pl.ds(base + b * CH, CH)], iv[b])
            pltpu.async_copy(table_hbm.at[iv[b]], rows[b], g[b])

        def body(j, carry):
            for b in range(2):
                i = 2 * j + b
                off = base + i * CH
                pltpu.make_async_copy(table_hbm.at[iv[b]], rows[b], g[b]).wait()
                out_cp = pltpu.make_async_copy(
                    rows[b], _out_slice(out_hbm, off), o[b])
                out_cp.start()

                @pl.when(i + 2 < n_chunks)
                def _():
                    pltpu.sync_copy(
                        idx_hbm.at[pl.ds(off + 2 * CH, CH)], iv[b])
                    out_cp.wait()
                    pltpu.async_copy(table_hbm.at[iv[b]], rows[b], g[b])

            return carry

        lax.fori_loop(0, n_chunks // 2, body, 0)

        for b in range(2):
            off = base + (n_chunks - 2 + b) * CH
            pltpu.make_async_copy(
                rows[b], _out_slice(out_hbm, off), o[b]).wait()

    return k


@functools.lru_cache(maxsize=None)
def _xpose_call(B0, H, E, BB):
    Q = 128 // E
    grid = (H, B0 // BB)

    def body(in_ref, out_ref):
        t = jnp.transpose(in_ref[0])
        out_ref[0] = jnp.concatenate(
            [t[q * E:(q + 1) * E] for q in range(Q)], axis=1)

    return pl.pallas_call(
        body,
        grid=grid,
        in_specs=[pl.BlockSpec((1, BB // Q, 128), lambda hi, bi: (hi, bi, 0))],
        out_specs=pl.BlockSpec((1, E, BB), lambda hi, bi: (hi, 0, bi)),
        out_shape=jax.ShapeDtypeStruct((H, E, B0), jnp.float32),
    )


def kernel(x, table):
    B0, H = x.shape
    E = table.shape[1]
    BB = 8192
    flat = jnp.transpose(x).reshape(B0 * H)
    g = _gather_call(B0 * H, E, 1024, BB, B0)(flat, table)
    gv = g.reshape(H, B0 * E // 128, 128)
    ot = _xpose_call(B0, H, E, BB)(gv)
    return jnp.transpose(ot, (2, 0, 1))


# speedup vs baseline: 1.1196x; 1.1196x over previous
import functools

import jax
import jax.numpy as jnp
from jax import lax
from jax.experimental import pallas as pl
from jax.experimental.pallas import tpu as pltpu
from jax.experimental.pallas import tpu_sc as plsc

_NC = 2
_NS = 16
_NW = _NC * _NS


@functools.lru_cache(maxsize=None)
def _gather_call(B, E, CH, BB, HW):
    per_w = B // _NW
    n_chunks = per_w // CH
    assert n_chunks >= 4 and n_chunks % 2 == 0
    mesh = plsc.VectorSubcoreMesh(core_axis_name="c", subcore_axis_name="s")

    Q = 128 // E
    YB = BB // Q

    def _out_slice(out_hbm, off):
        h = off // HW
        bo = off % HW
        blk = bo // BB
        t = bo % BB
        q = t // YB
        y0 = t % YB
        u0 = h * (HW // Q) + blk * YB + y0
        return out_hbm.at[pl.ds(u0, CH), pl.ds(q * E, E)]

    @functools.partial(
        pl.kernel,
        mesh=mesh,
        out_type=jax.ShapeDtypeStruct((B // Q, 128), jnp.float32),
        scratch_types=[
            pltpu.VMEM((CH,), jnp.int32),
            pltpu.VMEM((CH,), jnp.int32),
            pltpu.VMEM((CH, E), jnp.float32),
            pltpu.VMEM((CH, E), jnp.float32),
            pltpu.SemaphoreType.DMA,
            pltpu.SemaphoreType.DMA,
            pltpu.SemaphoreType.DMA,
            pltpu.SemaphoreType.DMA,
        ],
        compiler_params=pltpu.CompilerParams(use_tc_tiling_on_sc=False),
    )
    def k(idx_hbm, table_hbm, out_hbm, iv0, iv1, r0, r1, g0, g1, o0, o1):
        wid = lax.axis_index("s") * _NC + lax.axis_index("c")
        base = wid * per_w
        iv = (iv0, iv1)
        rows = (r0, r1)
        g = (g0, g1)
        o = (o0, o1)

        for b in range(2):
            pltpu.sync_copy(idx_hbm.at[pl.ds(base + b * CH, CH)], iv[b])
            pltpu.async_copy(table_hbm.at[iv[b]], rows[b], g[b])

        def body(j, carry):
            for b in range(2):
                i = 2 * j + b
                off = base + i * CH
                pltpu.make_async_copy(table_hbm.at[iv[b]], rows[b], g[b]).wait()
                out_cp = pltpu.make_async_copy(
                    rows[b], _out_slice(out_hbm, off), o[b])
                out_cp.start()

                @pl.when(i + 2 < n_chunks)
                def _():
                    pltpu.sync_copy(
                        idx_hbm.at[pl.ds(off + 2 * CH, CH)], iv[b])
                    out_cp.wait()
                    pltpu.async_copy(table_hbm.at[iv[b]], rows[b], g[b])

            return carry

        lax.fori_loop(0, n_chunks // 2, body, 0)

        for b in range(2):
            off = base + (n_chunks - 2 + b) * CH
            pltpu.make_async_copy(
                rows[b], _out_slice(out_hbm, off), o[b]).wait()

    return k


@functools.lru_cache(maxsize=None)
def _xpose_call(B0, H, E, BB):
    Q = 128 // E
    grid = (H, B0 // BB)

    def body(in_ref, out_ref):
        t = jnp.transpose(in_ref[0])
        out_ref[0] = jnp.concatenate(
            [t[q * E:(q + 1) * E] for q in range(Q)], axis=1)

    return pl.pallas_call(
        body,
        grid=grid,
        in_specs=[pl.BlockSpec((1, BB // Q, 128), lambda hi, bi: (hi, bi, 0))],
        out_specs=pl.BlockSpec((1, E, BB), lambda hi, bi: (hi, 0, bi)),
        out_shape=jax.ShapeDtypeStruct((H, E, B0), jnp.float32),
    )


def kernel(x, table):
    B0, H = x.shape
    E = table.shape[1]
    BB = 16384
    flat = jnp.transpose(x).reshape(B0 * H)
    g = _gather_call(B0 * H, E, 1024, BB, B0)(flat, table)
    gv = g.reshape(H, B0 * E // 128, 128)
    ot = _xpose_call(B0, H, E, BB)(gv)
    return jnp.transpose(ot, (2, 0, 1))


# speedup vs baseline: 1.3264x; 1.1847x over previous
import functools

import jax
import jax.numpy as jnp
from jax import lax
from jax.experimental import pallas as pl
from jax.experimental.pallas import tpu as pltpu
from jax.experimental.pallas import tpu_sc as plsc

_NC = 2
_NS = 16
_NW = _NC * _NS


@functools.lru_cache(maxsize=None)
def _gather_call(B, E, CH, BB, HW, TW):
    per_w = B // _NW
    n_chunks = per_w // CH
    assert n_chunks >= 4 and n_chunks % 2 == 0
    mesh = plsc.VectorSubcoreMesh(core_axis_name="c", subcore_axis_name="s")

    Q = 128 // E
    YB = BB // Q

    def _out_slice(out_hbm, off):
        h = off // HW
        bo = off % HW
        blk = bo // BB
        t = bo % BB
        q = t // YB
        y0 = t % YB
        u0 = h * (HW // Q) + blk * YB + y0
        return out_hbm.at[pl.ds(u0, CH), pl.ds(q * E, E)]

    @functools.partial(
        pl.kernel,
        mesh=mesh,
        out_type=jax.ShapeDtypeStruct((B // Q, 128), jnp.float32),
        scratch_types=[
            pltpu.VMEM((CH,), jnp.int32),
            pltpu.VMEM((CH,), jnp.int32),
            pltpu.VMEM((CH, E), jnp.float32),
            pltpu.VMEM((CH, E), jnp.float32),
            pltpu.SemaphoreType.DMA,
            pltpu.SemaphoreType.DMA,
            pltpu.SemaphoreType.DMA,
            pltpu.SemaphoreType.DMA,
        ],
        compiler_params=pltpu.CompilerParams(use_tc_tiling_on_sc=False),
    )
    def k(idx_hbm, table_hbm, out_hbm, iv0, iv1, r0, r1, g0, g1, o0, o1):
        wid = lax.axis_index("s") * _NC + lax.axis_index("c")
        base = wid * per_w
        iv = (iv0, iv1)
        rows = (r0, r1)
        g = (g0, g1)
        o = (o0, o1)

        tq_sh = (TW // Q).bit_length() - 1
        q_sh = Q.bit_length() - 1

        def load_idx(b, off):
            pltpu.sync_copy(idx_hbm.at[pl.ds(off, CH)], iv[b])

            def remap(v, carry):
                r = iv[b][pl.ds(v * 16, 16)]
                t = jnp.bitwise_and(r, TW - 1)
                iv[b][pl.ds(v * 16, 16)] = (
                    r - t
                    + jnp.left_shift(jnp.bitwise_and(t, TW // Q - 1), q_sh)
                    + jnp.right_shift(t, tq_sh))
                return carry

            lax.fori_loop(0, CH // 16, remap, 0)

        for b in range(2):
            load_idx(b, base + b * CH)
            pltpu.async_copy(table_hbm.at[iv[b]], rows[b], g[b])

        def body(j, carry):
            for b in range(2):
                i = 2 * j + b
                off = base + i * CH
                pltpu.make_async_copy(table_hbm.at[iv[b]], rows[b], g[b]).wait()
                out_cp = pltpu.make_async_copy(
                    rows[b], _out_slice(out_hbm, off), o[b])
                out_cp.start()

                @pl.when(i + 2 < n_chunks)
                def _():
                    load_idx(b, off + 2 * CH)
                    out_cp.wait()
                    pltpu.async_copy(table_hbm.at[iv[b]], rows[b], g[b])

            return carry

        lax.fori_loop(0, n_chunks // 2, body, 0)

        for b in range(2):
            off = base + (n_chunks - 2 + b) * CH
            pltpu.make_async_copy(
                rows[b], _out_slice(out_hbm, off), o[b]).wait()

    return k


@functools.lru_cache(maxsize=None)
def _tprep_call(V, E, W):
    Q = 128 // E
    WQ = W // Q
    nblk = (V + W - 1) // W

    def body(in_ref, out_ref):
        m = jnp.concatenate(
            [in_ref[:, q * WQ:(q + 1) * WQ] for q in range(Q)], axis=0)
        out_ref[...] = jnp.transpose(m)

    return pl.pallas_call(
        body,
        grid=(nblk,),
        in_specs=[pl.BlockSpec((E, W), lambda wi: (0, wi))],
        out_specs=pl.BlockSpec((WQ, 128), lambda wi: (wi, 0)),
        out_shape=jax.ShapeDtypeStruct((nblk * WQ, 128), jnp.float32),
    )


@functools.lru_cache(maxsize=None)
def _xpose_call(B0, H, E, BB):
    Q = 128 // E
    grid = (H, B0 // BB)

    def body(in_ref, out_ref):
        t = jnp.transpose(in_ref[0])
        out_ref[0] = jnp.concatenate(
            [t[q * E:(q + 1) * E] for q in range(Q)], axis=1)

    return pl.pallas_call(
        body,
        grid=grid,
        in_specs=[pl.BlockSpec((1, BB // Q, 128), lambda hi, bi: (hi, bi, 0))],
        out_specs=pl.BlockSpec((1, E, BB), lambda hi, bi: (hi, 0, bi)),
        out_shape=jax.ShapeDtypeStruct((H, E, B0), jnp.float32),
    )


def kernel(x, table):
    B0, H = x.shape
    V, E = table.shape
    BB = 16384
    TW = 2048
    tp = _tprep_call(V, E, TW)(jnp.transpose(table))
    tv = tp.reshape(tp.shape[0] * (128 // E), E)
    flat = jnp.transpose(x).reshape(B0 * H)
    g = _gather_call(B0 * H, E, 1024, BB, B0, TW)(flat, tv)
    gv = g.reshape(H, B0 * E // 128, 128)
    ot = _xpose_call(B0, H, E, BB)(gv)
    return jnp.transpose(ot, (2, 0, 1))


# speedup vs baseline: 1.3978x; 1.0538x over previous
import functools

import jax
import jax.numpy as jnp
from jax import lax
from jax.experimental import pallas as pl
from jax.experimental.pallas import tpu as pltpu
from jax.experimental.pallas import tpu_sc as plsc

_NC = 2
_NS = 16
_NW = _NC * _NS


@functools.lru_cache(maxsize=None)
def _gather_call(B, E, CH, BB, HW, TW):
    per_w = B // _NW
    n_chunks = per_w // CH
    assert n_chunks >= 4 and n_chunks % 2 == 0
    mesh = plsc.VectorSubcoreMesh(core_axis_name="c", subcore_axis_name="s")

    Q = 128 // E
    YB = BB // Q

    def _out_slice(out_hbm, off):
        h = off // HW
        bo = off % HW
        blk = bo // BB
        t = bo % BB
        q = t // YB
        y0 = t % YB
        u0 = h * (HW // Q) + blk * YB + y0
        return out_hbm.at[pl.ds(u0, CH), pl.ds(q * E, E)]

    @functools.partial(
        pl.kernel,
        mesh=mesh,
        out_type=jax.ShapeDtypeStruct((B // Q, 128), jnp.float32),
        scratch_types=[
            pltpu.VMEM((CH,), jnp.int32),
            pltpu.VMEM((CH,), jnp.int32),
            pltpu.VMEM((CH, E), jnp.float32),
            pltpu.VMEM((CH, E), jnp.float32),
            pltpu.SemaphoreType.DMA,
            pltpu.SemaphoreType.DMA,
            pltpu.SemaphoreType.DMA,
            pltpu.SemaphoreType.DMA,
        ],
        compiler_params=pltpu.CompilerParams(use_tc_tiling_on_sc=False),
    )
    def k(idx_hbm, table_hbm, out_hbm, iv0, iv1, r0, r1, g0, g1, o0, o1):
        wid = lax.axis_index("s") * _NC + lax.axis_index("c")
        base = wid * per_w
        iv = (iv0, iv1)
        rows = (r0, r1)
        g = (g0, g1)
        o = (o0, o1)

        tq_sh = (TW // Q).bit_length() - 1
        q_sh = Q.bit_length() - 1

        def load_idx(b, off):
            pltpu.sync_copy(idx_hbm.at[pl.ds(off, CH)], iv[b])

            def remap(v, carry):
                r = iv[b][pl.ds(v * 16, 16)]
                t = jnp.bitwise_and(r, TW - 1)
                iv[b][pl.ds(v * 16, 16)] = (
                    r - t
                    + jnp.left_shift(jnp.bitwise_and(t, TW // Q - 1), q_sh)
                    + jnp.right_shift(t, tq_sh))
                return carry

            lax.fori_loop(0, CH // 16, remap, 0)

        for b in range(2):
            load_idx(b, base + b * CH)
            pltpu.async_copy(table_hbm.at[iv[b]], rows[b], g[b])

        def body(j, carry):
            for b in range(2):
                i = 2 * j + b
                off = base + i * CH
                pltpu.make_async_copy(table_hbm.at[iv[b]], rows[b], g[b]).wait()
                out_cp = pltpu.make_async_copy(
                    rows[b], _out_slice(out_hbm, off), o[b])
                out_cp.start()

                @pl.when(i + 2 < n_chunks)
                def _():
                    load_idx(b, off + 2 * CH)
                    out_cp.wait()
                    pltpu.async_copy(table_hbm.at[iv[b]], rows[b], g[b])

            return carry

        lax.fori_loop(0, n_chunks // 2, body, 0)

        for b in range(2):
            off = base + (n_chunks - 2 + b) * CH
            pltpu.make_async_copy(
                rows[b], _out_slice(out_hbm, off), o[b]).wait()

    return k


@functools.lru_cache(maxsize=None)
def _tprep_call(V, E, W):
    Q = 128 // E
    WQ = W // Q
    nblk = (V + W - 1) // W

    def body(in_ref, out_ref):
        m = jnp.concatenate(
            [in_ref[:, q * WQ:(q + 1) * WQ] for q in range(Q)], axis=0)
        out_ref[...] = jnp.transpose(m)

    return pl.pallas_call(
        body,
        grid=(nblk,),
        in_specs=[pl.BlockSpec((E, W), lambda wi: (0, wi))],
        out_specs=pl.BlockSpec((WQ, 128), lambda wi: (wi, 0)),
        out_shape=jax.ShapeDtypeStruct((nblk * WQ, 128), jnp.float32),
    )


@functools.lru_cache(maxsize=None)
def _xpose_call(B0, H, E, BB):
    Q = 128 // E
    grid = (H, B0 // BB)

    HB = 2
    grid = (H // HB, B0 // BB)

    def body(in_ref, out_ref):
        for hh in range(HB):
            t = jnp.transpose(in_ref[hh])
            out_ref[hh] = jnp.concatenate(
                [t[q * E:(q + 1) * E] for q in range(Q)], axis=1)

    return pl.pallas_call(
        body,
        grid=grid,
        in_specs=[pl.BlockSpec((HB, BB // Q, 128), lambda hi, bi: (hi, bi, 0))],
        out_specs=pl.BlockSpec((HB, E, BB), lambda hi, bi: (hi, 0, bi)),
        out_shape=jax.ShapeDtypeStruct((H, E, B0), jnp.float32),
    )


def kernel(x, table):
    B0, H = x.shape
    V, E = table.shape
    BB = 16384
    TW = 2048
    tp = _tprep_call(V, E, TW)(jnp.transpose(table))
    tv = tp.reshape(tp.shape[0] * (128 // E), E)
    flat = jnp.transpose(x).reshape(B0 * H)
    g = _gather_call(B0 * H, E, 1024, BB, B0, TW)(flat, tv)
    gv = g.reshape(H, B0 * E // 128, 128)
    ot = _xpose_call(B0, H, E, BB)(gv)
    return jnp.transpose(ot, (2, 0, 1))


# speedup vs baseline: 1.3986x; 1.0006x over previous
import functools

import jax
import jax.numpy as jnp
from jax import lax
from jax.experimental import pallas as pl
from jax.experimental.pallas import tpu as pltpu
from jax.experimental.pallas import tpu_sc as plsc

_NC = 2
_NS = 16
_NW = _NC * _NS


@functools.lru_cache(maxsize=None)
def _gather_call(B, E, CH, BB, HW, TW):
    per_w = B // _NW
    n_chunks = per_w // CH
    assert n_chunks >= 4 and n_chunks % 2 == 0
    mesh = plsc.VectorSubcoreMesh(core_axis_name="c", subcore_axis_name="s")

    Q = 128 // E
    YB = BB // Q

    def _out_slice(out_hbm, off):
        h = off // HW
        bo = off % HW
        blk = bo // BB
        t = bo % BB
        q = t // YB
        y0 = t % YB
        u0 = h * (HW // Q) + blk * YB + y0
        return out_hbm.at[pl.ds(u0, CH), pl.ds(q * E, E)]

    @functools.partial(
        pl.kernel,
        mesh=mesh,
        out_type=jax.ShapeDtypeStruct((B // Q, 128), jnp.float32),
        scratch_types=[
            pltpu.VMEM((CH,), jnp.int32),
            pltpu.VMEM((CH,), jnp.int32),
            pltpu.VMEM((CH, E), jnp.float32),
            pltpu.VMEM((CH, E), jnp.float32),
            pltpu.SemaphoreType.DMA,
            pltpu.SemaphoreType.DMA,
            pltpu.SemaphoreType.DMA,
            pltpu.SemaphoreType.DMA,
        ],
        compiler_params=pltpu.CompilerParams(use_tc_tiling_on_sc=False),
    )
    def k(idx_hbm, table_hbm, out_hbm, iv0, iv1, r0, r1, g0, g1, o0, o1):
        wid = lax.axis_index("s") * _NC + lax.axis_index("c")
        base = wid * per_w
        iv = (iv0, iv1)
        rows = (r0, r1)
        g = (g0, g1)
        o = (o0, o1)

        tq_sh = (TW // Q).bit_length() - 1
        q_sh = Q.bit_length() - 1

        def load_idx(b, off):
            pltpu.sync_copy(idx_hbm.at[pl.ds(off, CH)], iv[b])

            def remap(v, carry):
                r = iv[b][pl.ds(v * 16, 16)]
                t = jnp.bitwise_and(r, TW - 1)
                iv[b][pl.ds(v * 16, 16)] = (
                    r - t
                    + jnp.left_shift(jnp.bitwise_and(t, TW // Q - 1), q_sh)
                    + jnp.right_shift(t, tq_sh))
                return carry

            lax.fori_loop(0, CH // 16, remap, 0)

        for b in range(2):
            load_idx(b, base + b * CH)
            pltpu.async_copy(table_hbm.at[iv[b]], rows[b], g[b])

        def body(j, carry):
            for b in range(2):
                i = 2 * j + b
                off = base + i * CH
                pltpu.make_async_copy(table_hbm.at[iv[b]], rows[b], g[b]).wait()
                out_cp = pltpu.make_async_copy(
                    rows[b], _out_slice(out_hbm, off), o[b])
                out_cp.start()

                @pl.when(i + 2 < n_chunks)
                def _():
                    load_idx(b, off + 2 * CH)
                    out_cp.wait()
                    pltpu.async_copy(table_hbm.at[iv[b]], rows[b], g[b])

            return carry

        lax.fori_loop(0, n_chunks // 2, body, 0)

        for b in range(2):
            off = base + (n_chunks - 2 + b) * CH
            pltpu.make_async_copy(
                rows[b], _out_slice(out_hbm, off), o[b]).wait()

    return k


@functools.lru_cache(maxsize=None)
def _tprep_call(V, E, W):
    Q = 128 // E
    WQ = W // Q
    nblk = (V + W - 1) // W

    def body(in_ref, out_ref):
        m = jnp.concatenate(
            [in_ref[:, q * WQ:(q + 1) * WQ] for q in range(Q)], axis=0)
        out_ref[...] = jnp.transpose(m)

    return pl.pallas_call(
        body,
        grid=(nblk,),
        in_specs=[pl.BlockSpec((E, W), lambda wi: (0, wi))],
        out_specs=pl.BlockSpec((WQ, 128), lambda wi: (wi, 0)),
        out_shape=jax.ShapeDtypeStruct((nblk * WQ, 128), jnp.float32),
    )


@functools.lru_cache(maxsize=None)
def _xpose_call(B0, H, E, BB):
    Q = 128 // E
    HB = 2
    grid = (H // HB, B0 // BB)

    def body(in_ref, out_ref):
        for hh in range(HB):
            t = jnp.transpose(in_ref[hh])
            out_ref[hh] = jnp.concatenate(
                [t[q * E:(q + 1) * E] for q in range(Q)], axis=1)

    return pl.pallas_call(
        body,
        grid=grid,
        in_specs=[pl.BlockSpec((HB, BB // Q, 128), lambda hi, bi: (hi, bi, 0))],
        out_specs=pl.BlockSpec((HB, E, BB), lambda hi, bi: (hi, 0, bi)),
        out_shape=jax.ShapeDtypeStruct((H, E, B0), jnp.float32),
    )


def kernel(x, table):
    B0, H = x.shape
    V, E = table.shape
    BB = 16384
    TW = 2048
    tp = _tprep_call(V, E, TW)(jnp.transpose(table))
    tv = tp.reshape(tp.shape[0] * (128 // E), E)
    flat = jnp.transpose(x).reshape(B0 * H)
    g = _gather_call(B0 * H, E, 1024, BB, B0, TW)(flat, tv)
    gv = g.reshape(H, B0 * E // 128, 128)
    ot = _xpose_call(B0, H, E, BB)(gv)
    return jnp.transpose(ot, (2, 0, 1))
